# baseline (speedup 1.0000x reference)
import math

import jax
import jax.numpy as jnp
from jax.experimental import pallas as pl
from jax.experimental.pallas import tpu as pltpu
from jax.experimental.pallas import tpu_sc as plsc

B, KV, D, H, DH, DFF = 16, 2048, 2048, 16, 128, 8192
TOPK, HSEL = 2048, 8
EPS = 1e-5


def _topk_sel(logits, k, idx_bits):
    r, c = logits.shape
    bits = jax.lax.bitcast_convert_type(logits, jnp.int32)
    keys = jnp.where(bits < 0, bits ^ jnp.int32(0x7FFFFFFF), bits)
    kk = jnp.int32(k)
    cnt = jnp.sum((keys >= 0).astype(jnp.int32), axis=1, keepdims=True)
    t = jnp.where(cnt >= kk, jnp.zeros((r, 1), jnp.int32),
                  jnp.full((r, 1), -2147483648, jnp.int32))
    for bit in range(30, -1, -1):
        cand = t + jnp.int32(1 << bit)
        cnt = jnp.sum((keys >= cand).astype(jnp.int32), axis=1, keepdims=True)
        t = jnp.where(cnt >= kk, cand, t)
    gt = keys > t
    eq = keys == t
    need = kk - jnp.sum(gt.astype(jnp.int32), axis=1, keepdims=True)
    iota = jax.lax.broadcasted_iota(jnp.int32, (r, c), 1)
    lim = jnp.zeros((r, 1), jnp.int32)
    for bit in range(idx_bits - 1, -1, -1):
        cand = lim + jnp.int32(1 << bit)
        cnt = jnp.sum((eq & (iota < cand)).astype(jnp.int32), axis=1,
                      keepdims=True)
        lim = jnp.where(cnt <= need - 1, cand, lim)
    return gt | (eq & (iota <= lim))


def _router_kernel(x_ref, w_ref, hw_ref, o_ref, hidx_ref):
    o_ref[...] = jax.lax.dot_general(
        x_ref[:, 0, :], w_ref[...], (((1,), (0,)), ((), ())),
        preferred_element_type=jnp.float32)

    @pl.when(pl.program_id(0) == pl.num_programs(0) - 1)
    def _():
        head_logits = jax.lax.dot_general(
            x_ref[:, 0, :], hw_ref[...], (((1,), (0,)), ((), ())),
            preferred_element_type=jnp.float32)
        hsel = _topk_sel(head_logits, HSEL, 4)
        ri = jax.lax.broadcasted_iota(jnp.int32, (H, H), 0)
        ci = jax.lax.broadcasted_iota(jnp.int32, (H, H), 1)
        ut = (ri <= ci).astype(jnp.float32)
        rank = jax.lax.dot_general(
            hsel.astype(jnp.float32), ut, (((1,), (0,)), ((), ())),
            preferred_element_type=jnp.float32)
        iota_h = jax.lax.broadcasted_iota(jnp.int32, (B, H), 1)
        cols = []
        for j in range(HSEL):
            hit = hsel & (rank == jnp.float32(j + 1))
            cols.append(jnp.sum(jnp.where(hit, iota_h, 0), axis=1,
                                keepdims=True))
        hidx_ref[...] = jnp.concatenate(cols, axis=1)


def _qkv_kernel(x_ref, w_ref, b_ref, o_ref):
    o_ref[...] = jax.lax.dot_general(
        x_ref[:, 0, :], w_ref[...], (((1,), (1,)), ((), ())),
        preferred_element_type=jnp.float32) + b_ref[...]


_SC_L = 16
_SC_NV = DFF // _SC_L

_GDN = jax.lax.GatherDimensionNumbers(
    offset_dims=(), collapsed_slice_dims=(0,), start_index_map=(0,))


def _lane_sum(x):
    for stride in (1, 2, 4, 8):
        idx = jax.lax.iota(jnp.int32, _SC_L) ^ stride
        x = x + jax.lax.gather(
            x, idx[:, None], _GDN, slice_sizes=(1,),
            mode=jax.lax.GatherScatterMode.PROMISE_IN_BOUNDS)
    return x


def _mask_sc_body(logits_hbm, mask_hbm, row_v, keys_v):
    c = jax.lax.axis_index("c")
    s = jax.lax.axis_index("s")
    wid = s * 2 + c

    @pl.when(wid < B)
    def _():
        pltpu.sync_copy(logits_hbm.at[wid], row_v)

        def mk(i, carry):
            v = row_v[pl.ds(i * _SC_L, _SC_L)]
            bits = jax.lax.bitcast_convert_type(v, jnp.int32)
            keys_v[pl.ds(i * _SC_L, _SC_L)] = jnp.where(
                bits < 0, bits ^ jnp.int32(0x7FFFFFFF), bits)
            return carry

        jax.lax.fori_loop(0, _SC_NV, mk, jnp.int32(0), unroll=8)

        def splat(v):
            return jnp.full((_SC_L,), v, jnp.int32)

        def count_ge(cand_v):
            def cb(i, acc):
                kv = keys_v[pl.ds(i * _SC_L, _SC_L)]
                return acc + jnp.where(kv >= cand_v, 1, 0)
            acc = jax.lax.fori_loop(0, _SC_NV, cb,
                                    jnp.zeros((_SC_L,), jnp.int32), unroll=8)
            return _lane_sum(acc)

        kk = splat(TOPK)
        zero = splat(0)
        t = jnp.where(count_ge(zero) >= kk, zero, splat(-2147483648))
        for bit in range(30, -1, -1):
            cand = t + splat(1 << bit)
            t = jnp.where(count_ge(cand) >= kk, cand, t)
        need = kk - count_ge(t + splat(1))

        def count_eq_lt(cl_v):
            def cb(i, acc):
                kv = keys_v[pl.ds(i * _SC_L, _SC_L)]
                idx = jax.lax.iota(jnp.int32, _SC_L) + i * _SC_L
                m = (kv == t) & (idx < cl_v)
                return acc + jnp.where(m, 1, 0)
            acc = jax.lax.fori_loop(0, _SC_NV, cb,
                                    jnp.zeros((_SC_L,), jnp.int32), unroll=8)
            return _lane_sum(acc)

        lim = splat(0)
        for bit in range(12, -1, -1):
            cand = lim + splat(1 << bit)
            lim = jnp.where(count_eq_lt(cand) <= need - splat(1), cand, lim)

        def wb(i, carry):
            kv = keys_v[pl.ds(i * _SC_L, _SC_L)]
            idx = jax.lax.iota(jnp.int32, _SC_L) + i * _SC_L
            sel = (kv > t) | ((kv == t) & (idx <= lim))
            row_v[pl.ds(i * _SC_L, _SC_L)] = jnp.where(
                sel, jnp.float32(1.0), jnp.float32(0.0))
            return carry

        jax.lax.fori_loop(0, _SC_NV, wb, jnp.int32(0), unroll=8)
        pltpu.sync_copy(row_v, mask_hbm.at[wid])


def _mask_sc(mlp_logits):
    return pl.kernel(
        _mask_sc_body,
        out_type=jax.ShapeDtypeStruct((B, DFF), jnp.float32),
        mesh=plsc.VectorSubcoreMesh(core_axis_name="c", subcore_axis_name="s",
                                    num_cores=2, num_subcores=16),
        scratch_types=[
            pltpu.VMEM((DFF,), jnp.float32),
            pltpu.VMEM((DFF,), jnp.int32),
        ],
    )(mlp_logits)


def _one_head(qkv_ref, k_ref, v_ref, b, h):
    q = qkv_ref[b, pl.ds(h, 1), :]
    kn = qkv_ref[b, pl.ds(H + h, 1), :]
    vn = qkv_ref[b, pl.ds(2 * H + h, 1), :]
    scale = jnp.float32(1.0 / math.sqrt(DH))
    s = jax.lax.dot_general(
        q, k_ref[0, 0], (((1,), (1,)), ((), ())),
        preferred_element_type=jnp.float32) * scale
    sn = jnp.sum(q * kn, axis=1, keepdims=True) * scale
    m = jnp.maximum(jnp.max(s, axis=1, keepdims=True), sn)
    p = jnp.exp(s - m)
    pn = jnp.exp(sn - m)
    denom = jnp.sum(p, axis=1, keepdims=True) + pn
    o = jax.lax.dot_general(
        p, v_ref[0, 0], (((1,), (0,)), ((), ())),
        preferred_element_type=jnp.float32)
    return (o + pn * vn) / denom


def _attn_kernel(idx_ref, k0_ref, v0_ref, k1_ref, v1_ref, k2_ref, v2_ref,
                 k3_ref, v3_ref, k4_ref, v4_ref, k5_ref, v5_ref, k6_ref,
                 v6_ref, k7_ref, v7_ref, qkv_ref, wout_ref, bout_ref,
                 res_ref, nw_ref, nb_ref, rout_ref, hn_ref):
    b = pl.program_id(0)
    kvs = ((k0_ref, v0_ref), (k1_ref, v1_ref), (k2_ref, v2_ref),
           (k3_ref, v3_ref), (k4_ref, v4_ref), (k5_ref, v5_ref),
           (k6_ref, v6_ref), (k7_ref, v7_ref))
    mixer = bout_ref[...]
    for u, (kr, vr) in enumerate(kvs):
        h = idx_ref[b, u]
        attn_u = _one_head(qkv_ref, kr, vr, b, h)
        wblk = wout_ref[:, pl.ds(pl.multiple_of(h * DH, DH), DH)]
        mixer = mixer + jax.lax.dot_general(
            attn_u, wblk, (((1,), (1,)), ((), ())),
            preferred_element_type=jnp.float32)
    rout = mixer + res_ref[0]
    rout_ref[0] = rout
    mu = jnp.mean(rout, axis=1, keepdims=True)
    xc = rout - mu
    var = jnp.mean(xc * xc, axis=1, keepdims=True)
    hn_ref[0] = xc * jax.lax.rsqrt(var + EPS) * nw_ref[...] + nb_ref[...]


def _mlp_kernel(hn_ref, w1_ref, b1_ref, w2_ref, mask_ref, b2_ref, o_ref):
    pre = jax.lax.dot_general(
        hn_ref[:, 0, :], w1_ref[...], (((1,), (1,)), ((), ())),
        preferred_element_type=jnp.float32) + b1_ref[...]
    act = jax.nn.gelu(pre) * mask_ref[...]
    part = jax.lax.dot_general(
        act, w2_ref[...], (((1,), (1,)), ((), ())),
        preferred_element_type=jnp.float32)

    @pl.when(pl.program_id(0) == 0)
    def _():
        o_ref[:, 0, :] = part + b2_ref[...]

    @pl.when(pl.program_id(0) > 0)
    def _():
        o_ref[:, 0, :] = o_ref[:, 0, :] + part


def kernel(hidden_states, residual, k_cache, v_cache, Wqkv, bqkv, Wout, bout,
           norm2_w, norm2_b, fc1_w, fc1_b, fc2_w, fc2_b, mlp_router_w,
           mha_router_w):
    cf = 2048
    mlp_logits, head_idx = pl.pallas_call(
        _router_kernel,
        grid=(DFF // cf,),
        in_specs=[
            pl.BlockSpec((B, 1, D), lambda i: (0, 0, 0)),
            pl.BlockSpec((D, cf), lambda i: (0, i)),
            pl.BlockSpec((D, H), lambda i: (0, 0)),
        ],
        out_specs=[
            pl.BlockSpec((B, cf), lambda i: (0, i)),
            pl.BlockSpec((B, HSEL), lambda i: (0, 0)),
        ],
        out_shape=[
            jax.ShapeDtypeStruct((B, DFF), jnp.float32),
            jax.ShapeDtypeStruct((B, HSEL), jnp.int32),
        ],
    )(hidden_states, mlp_router_w, mha_router_w)

    cq = 2048
    qkv = pl.pallas_call(
        _qkv_kernel,
        grid=(3 * D // cq,),
        in_specs=[
            pl.BlockSpec((B, 1, D), lambda i: (0, 0, 0)),
            pl.BlockSpec((cq, D), lambda i: (i, 0)),
            pl.BlockSpec((1, cq), lambda i: (0, i)),
        ],
        out_specs=pl.BlockSpec((B, cq), lambda i: (0, i)),
        out_shape=jax.ShapeDtypeStruct((B, 3 * D), jnp.float32),
    )(hidden_states, Wqkv, bqkv.reshape(1, 3 * D))

    mask = _mask_sc(mlp_logits)

    qkv_r = qkv.reshape(B, 3 * H, DH)

    def _kv_spec(u):
        return pl.BlockSpec((1, 1, KV, DH),
                            lambda b, idx, u=u: (b, idx[b, u], 0, 0))

    rout, hn = pl.pallas_call(
        _attn_kernel,
        grid_spec=pltpu.PrefetchScalarGridSpec(
            num_scalar_prefetch=1,
            grid=(B,),
            in_specs=(
                [s for u in range(HSEL) for s in (_kv_spec(u), _kv_spec(u))]
                + [
                    pl.BlockSpec((B, 3 * H, DH), lambda b, idx: (0, 0, 0)),
                    pl.BlockSpec((D, D), lambda b, idx: (0, 0)),
                    pl.BlockSpec((1, D), lambda b, idx: (0, 0)),
                    pl.BlockSpec((1, 1, D), lambda b, idx: (b, 0, 0)),
                    pl.BlockSpec((1, D), lambda b, idx: (0, 0)),
                    pl.BlockSpec((1, D), lambda b, idx: (0, 0)),
                ]
            ),
            out_specs=[
                pl.BlockSpec((1, 1, D), lambda b, idx: (b, 0, 0)),
                pl.BlockSpec((1, 1, D), lambda b, idx: (b, 0, 0)),
            ],
        ),
        out_shape=[
            jax.ShapeDtypeStruct((B, 1, D), jnp.float32),
            jax.ShapeDtypeStruct((B, 1, D), jnp.float32),
        ],
    )(head_idx, *([k_cache, v_cache] * HSEL), qkv_r, Wout,
      bout.reshape(1, D), residual, norm2_w.reshape(1, D),
      norm2_b.reshape(1, D))

    cm = 1024
    mlp_out = pl.pallas_call(
        _mlp_kernel,
        grid=(DFF // cm,),
        in_specs=[
            pl.BlockSpec((B, 1, D), lambda i: (0, 0, 0)),
            pl.BlockSpec((cm, D), lambda i: (i, 0)),
            pl.BlockSpec((1, cm), lambda i: (0, i)),
            pl.BlockSpec((D, cm), lambda i: (0, i)),
            pl.BlockSpec((B, cm), lambda i: (0, i)),
            pl.BlockSpec((1, D), lambda i: (0, 0)),
        ],
        out_specs=pl.BlockSpec((B, 1, D), lambda i: (0, 0, 0)),
        out_shape=jax.ShapeDtypeStruct((B, 1, D), jnp.float32),
    )(hn, fc1_w, fc1_b.reshape(1, DFF), fc2_w, mask, fc2_b.reshape(1, D))

    return (mlp_out, rout)


# speedup vs baseline: 1.0464x; 1.0464x over previous
import math

import jax
import jax.numpy as jnp
from jax.experimental import pallas as pl
from jax.experimental.pallas import tpu as pltpu
from jax.experimental.pallas import tpu_sc as plsc

B, KV, D, H, DH, DFF = 16, 2048, 2048, 16, 128, 8192
TOPK, HSEL = 2048, 8
EPS = 1e-5


def _topk_sel(logits, k, idx_bits):
    r, c = logits.shape
    bits = jax.lax.bitcast_convert_type(logits, jnp.int32)
    keys = jnp.where(bits < 0, bits ^ jnp.int32(0x7FFFFFFF), bits)
    kk = jnp.int32(k)
    cnt = jnp.sum((keys >= 0).astype(jnp.int32), axis=1, keepdims=True)
    t = jnp.where(cnt >= kk, jnp.zeros((r, 1), jnp.int32),
                  jnp.full((r, 1), -2147483648, jnp.int32))
    for bit in range(30, -1, -1):
        cand = t + jnp.int32(1 << bit)
        cnt = jnp.sum((keys >= cand).astype(jnp.int32), axis=1, keepdims=True)
        t = jnp.where(cnt >= kk, cand, t)
    gt = keys > t
    eq = keys == t
    need = kk - jnp.sum(gt.astype(jnp.int32), axis=1, keepdims=True)
    iota = jax.lax.broadcasted_iota(jnp.int32, (r, c), 1)
    lim = jnp.zeros((r, 1), jnp.int32)
    for bit in range(idx_bits - 1, -1, -1):
        cand = lim + jnp.int32(1 << bit)
        cnt = jnp.sum((eq & (iota < cand)).astype(jnp.int32), axis=1,
                      keepdims=True)
        lim = jnp.where(cnt <= need - 1, cand, lim)
    return gt | (eq & (iota <= lim))


def _router_kernel(x_ref, w_ref, hw_ref, o_ref, hidx_ref):
    o_ref[...] = jax.lax.dot_general(
        x_ref[:, 0, :], w_ref[...], (((1,), (0,)), ((), ())),
        preferred_element_type=jnp.float32)

    @pl.when(pl.program_id(0) == pl.num_programs(0) - 1)
    def _():
        head_logits = jax.lax.dot_general(
            x_ref[:, 0, :], hw_ref[...], (((1,), (0,)), ((), ())),
            preferred_element_type=jnp.float32)
        hsel = _topk_sel(head_logits, HSEL, 4)
        ri = jax.lax.broadcasted_iota(jnp.int32, (H, H), 0)
        ci = jax.lax.broadcasted_iota(jnp.int32, (H, H), 1)
        ut = (ri <= ci).astype(jnp.float32)
        rank = jax.lax.dot_general(
            hsel.astype(jnp.float32), ut, (((1,), (0,)), ((), ())),
            preferred_element_type=jnp.float32)
        iota_h = jax.lax.broadcasted_iota(jnp.int32, (B, H), 1)
        cols = []
        for j in range(HSEL):
            hit = hsel & (rank == jnp.float32(j + 1))
            cols.append(jnp.sum(jnp.where(hit, iota_h, 0), axis=1,
                                keepdims=True))
        hidx_ref[...] = jnp.concatenate(cols, axis=1)


def _qkv_kernel(x_ref, w_ref, b_ref, o_ref):
    o_ref[...] = jax.lax.dot_general(
        x_ref[:, 0, :], w_ref[...], (((1,), (1,)), ((), ())),
        preferred_element_type=jnp.float32) + b_ref[...]


_SC_L = 16
_SC_NV = DFF // _SC_L

_GDN = jax.lax.GatherDimensionNumbers(
    offset_dims=(), collapsed_slice_dims=(0,), start_index_map=(0,))


def _lane_sum(x):
    for stride in (1, 2, 4, 8):
        idx = jax.lax.iota(jnp.int32, _SC_L) ^ stride
        x = x + jax.lax.gather(
            x, idx[:, None], _GDN, slice_sizes=(1,),
            mode=jax.lax.GatherScatterMode.PROMISE_IN_BOUNDS)
    return x


def _mask_sc_body(logits_hbm, mask_hbm, row_v, keys_v):
    c = jax.lax.axis_index("c")
    s = jax.lax.axis_index("s")
    wid = s * 2 + c

    @pl.when(wid < B)
    def _():
        pltpu.sync_copy(logits_hbm.at[wid], row_v)

        def mk(i, carry):
            v = row_v[pl.ds(i * _SC_L, _SC_L)]
            bits = jax.lax.bitcast_convert_type(v, jnp.int32)
            keys_v[pl.ds(i * _SC_L, _SC_L)] = jnp.where(
                bits < 0, bits ^ jnp.int32(0x7FFFFFFF), bits)
            return carry

        jax.lax.fori_loop(0, _SC_NV, mk, jnp.int32(0), unroll=8)

        def splat(v):
            return jnp.full((_SC_L,), v, jnp.int32)

        def count_ge(cand_v):
            def cb(i, acc):
                kv = keys_v[pl.ds(i * _SC_L, _SC_L)]
                return acc + jnp.where(kv >= cand_v, 1, 0)
            acc = jax.lax.fori_loop(0, _SC_NV, cb,
                                    jnp.zeros((_SC_L,), jnp.int32), unroll=8)
            return _lane_sum(acc)

        kk = splat(TOPK)
        zero = splat(0)
        t = jnp.where(count_ge(zero) >= kk, zero, splat(-2147483648))
        for bit in range(30, -1, -1):
            cand = t + splat(1 << bit)
            t = jnp.where(count_ge(cand) >= kk, cand, t)
        need = kk - count_ge(t + splat(1))

        def count_eq_lt(cl_v):
            def cb(i, acc):
                kv = keys_v[pl.ds(i * _SC_L, _SC_L)]
                idx = jax.lax.iota(jnp.int32, _SC_L) + i * _SC_L
                m = (kv == t) & (idx < cl_v)
                return acc + jnp.where(m, 1, 0)
            acc = jax.lax.fori_loop(0, _SC_NV, cb,
                                    jnp.zeros((_SC_L,), jnp.int32), unroll=8)
            return _lane_sum(acc)

        lim = splat(0)
        for bit in range(12, -1, -1):
            cand = lim + splat(1 << bit)
            lim = jnp.where(count_eq_lt(cand) <= need - splat(1), cand, lim)

        def wb(i, carry):
            kv = keys_v[pl.ds(i * _SC_L, _SC_L)]
            idx = jax.lax.iota(jnp.int32, _SC_L) + i * _SC_L
            sel = (kv > t) | ((kv == t) & (idx <= lim))
            row_v[pl.ds(i * _SC_L, _SC_L)] = jnp.where(
                sel, jnp.float32(1.0), jnp.float32(0.0))
            return carry

        jax.lax.fori_loop(0, _SC_NV, wb, jnp.int32(0), unroll=8)
        pltpu.sync_copy(row_v, mask_hbm.at[wid])


def _mask_sc(mlp_logits):
    return pl.kernel(
        _mask_sc_body,
        out_type=jax.ShapeDtypeStruct((B, DFF), jnp.float32),
        mesh=plsc.VectorSubcoreMesh(core_axis_name="c", subcore_axis_name="s",
                                    num_cores=2, num_subcores=16),
        scratch_types=[
            pltpu.VMEM((DFF,), jnp.float32),
            pltpu.VMEM((DFF,), jnp.int32),
        ],
    )(mlp_logits)


def _one_head(qkv_ref, k_ref, v_ref, b, h):
    q = qkv_ref[b, pl.ds(h, 1), :]
    kn = qkv_ref[b, pl.ds(H + h, 1), :]
    vn = qkv_ref[b, pl.ds(2 * H + h, 1), :]
    scale = jnp.float32(1.0 / math.sqrt(DH))
    s = jax.lax.dot_general(
        q, k_ref[0, 0], (((1,), (1,)), ((), ())),
        preferred_element_type=jnp.float32) * scale
    sn = jnp.sum(q * kn, axis=1, keepdims=True) * scale
    m = jnp.maximum(jnp.max(s, axis=1, keepdims=True), sn)
    p = jnp.exp(s - m)
    pn = jnp.exp(sn - m)
    denom = jnp.sum(p, axis=1, keepdims=True) + pn
    o = jax.lax.dot_general(
        p, v_ref[0, 0], (((1,), (0,)), ((), ())),
        preferred_element_type=jnp.float32)
    return (o + pn * vn) / denom


def _attn_kernel(idx_ref, k0_ref, v0_ref, k1_ref, v1_ref, k2_ref, v2_ref,
                 k3_ref, v3_ref, k4_ref, v4_ref, k5_ref, v5_ref, k6_ref,
                 v6_ref, k7_ref, v7_ref, qkv_ref, o_ref):
    b = pl.program_id(0)
    kvs = ((k0_ref, v0_ref), (k1_ref, v1_ref), (k2_ref, v2_ref),
           (k3_ref, v3_ref), (k4_ref, v4_ref), (k5_ref, v5_ref),
           (k6_ref, v6_ref), (k7_ref, v7_ref))
    for u, (kr, vr) in enumerate(kvs):
        o_ref[0, u] = _one_head(qkv_ref, kr, vr, b, idx_ref[b, u])


def _mix_kernel(attn_ref, hidx_ref, wout_ref, bout_ref, res_ref, nw_ref,
                nb_ref, rout_ref, hn_ref):
    hidx = hidx_ref[...]
    blocks = []
    for h in range(H):
        contrib = jnp.zeros((B, DH), jnp.float32)
        for j in range(HSEL):
            contrib = contrib + jnp.where(hidx[:, j:j + 1] == h,
                                          attn_ref[:, j, 0, :], 0.0)
        blocks.append(contrib)
    attn_full = jnp.concatenate(blocks, axis=1)
    mixer = jax.lax.dot_general(
        attn_full, wout_ref[...], (((1,), (1,)), ((), ())),
        preferred_element_type=jnp.float32) + bout_ref[...]
    rout = mixer + res_ref[:, 0, :]
    rout_ref[:, 0, :] = rout
    mu = jnp.mean(rout, axis=1, keepdims=True)
    xc = rout - mu
    var = jnp.mean(xc * xc, axis=1, keepdims=True)
    hn_ref[:, 0, :] = (xc * jax.lax.rsqrt(var + EPS) * nw_ref[...]
                       + nb_ref[...])


def _mlp_kernel(hn_ref, w1_ref, b1_ref, w2_ref, mask_ref, b2_ref, o_ref):
    pre = jax.lax.dot_general(
        hn_ref[:, 0, :], w1_ref[...], (((1,), (1,)), ((), ())),
        preferred_element_type=jnp.float32) + b1_ref[...]
    act = jax.nn.gelu(pre) * mask_ref[...]
    part = jax.lax.dot_general(
        act, w2_ref[...], (((1,), (1,)), ((), ())),
        preferred_element_type=jnp.float32)

    @pl.when(pl.program_id(0) == 0)
    def _():
        o_ref[:, 0, :] = part + b2_ref[...]

    @pl.when(pl.program_id(0) > 0)
    def _():
        o_ref[:, 0, :] = o_ref[:, 0, :] + part


def kernel(hidden_states, residual, k_cache, v_cache, Wqkv, bqkv, Wout, bout,
           norm2_w, norm2_b, fc1_w, fc1_b, fc2_w, fc2_b, mlp_router_w,
           mha_router_w):
    cf = 2048
    mlp_logits, head_idx = pl.pallas_call(
        _router_kernel,
        grid=(DFF // cf,),
        in_specs=[
            pl.BlockSpec((B, 1, D), lambda i: (0, 0, 0)),
            pl.BlockSpec((D, cf), lambda i: (0, i)),
            pl.BlockSpec((D, H), lambda i: (0, 0)),
        ],
        out_specs=[
            pl.BlockSpec((B, cf), lambda i: (0, i)),
            pl.BlockSpec((B, HSEL), lambda i: (0, 0)),
        ],
        out_shape=[
            jax.ShapeDtypeStruct((B, DFF), jnp.float32),
            jax.ShapeDtypeStruct((B, HSEL), jnp.int32),
        ],
    )(hidden_states, mlp_router_w, mha_router_w)

    cq = 2048
    qkv = pl.pallas_call(
        _qkv_kernel,
        grid=(3 * D // cq,),
        in_specs=[
            pl.BlockSpec((B, 1, D), lambda i: (0, 0, 0)),
            pl.BlockSpec((cq, D), lambda i: (i, 0)),
            pl.BlockSpec((1, cq), lambda i: (0, i)),
        ],
        out_specs=pl.BlockSpec((B, cq), lambda i: (0, i)),
        out_shape=jax.ShapeDtypeStruct((B, 3 * D), jnp.float32),
    )(hidden_states, Wqkv, bqkv.reshape(1, 3 * D))

    mask = _mask_sc(mlp_logits)

    qkv_r = qkv.reshape(B, 3 * H, DH)

    def _kv_spec(u):
        return pl.BlockSpec((1, 1, KV, DH),
                            lambda b, idx, u=u: (b, idx[b, u], 0, 0))

    attn_c = pl.pallas_call(
        _attn_kernel,
        grid_spec=pltpu.PrefetchScalarGridSpec(
            num_scalar_prefetch=1,
            grid=(B,),
            in_specs=(
                [s for u in range(HSEL) for s in (_kv_spec(u), _kv_spec(u))]
                + [pl.BlockSpec((B, 3 * H, DH), lambda b, idx: (0, 0, 0))]
            ),
            out_specs=pl.BlockSpec((1, HSEL, 1, DH),
                                   lambda b, idx: (b, 0, 0, 0)),
        ),
        out_shape=jax.ShapeDtypeStruct((B, HSEL, 1, DH), jnp.float32),
    )(head_idx, *([k_cache, v_cache] * HSEL), qkv_r)

    rout, hn = pl.pallas_call(
        _mix_kernel,
        in_specs=[
            pl.BlockSpec((B, HSEL, 1, DH), lambda: (0, 0, 0, 0)),
            pl.BlockSpec((B, HSEL), lambda: (0, 0)),
            pl.BlockSpec((D, D), lambda: (0, 0)),
            pl.BlockSpec((1, D), lambda: (0, 0)),
            pl.BlockSpec((B, 1, D), lambda: (0, 0, 0)),
            pl.BlockSpec((1, D), lambda: (0, 0)),
            pl.BlockSpec((1, D), lambda: (0, 0)),
        ],
        out_specs=[
            pl.BlockSpec((B, 1, D), lambda: (0, 0, 0)),
            pl.BlockSpec((B, 1, D), lambda: (0, 0, 0)),
        ],
        out_shape=[
            jax.ShapeDtypeStruct((B, 1, D), jnp.float32),
            jax.ShapeDtypeStruct((B, 1, D), jnp.float32),
        ],
    )(attn_c, head_idx, Wout, bout.reshape(1, D), residual,
      norm2_w.reshape(1, D), norm2_b.reshape(1, D))

    cm = 1024
    mlp_out = pl.pallas_call(
        _mlp_kernel,
        grid=(DFF // cm,),
        in_specs=[
            pl.BlockSpec((B, 1, D), lambda i: (0, 0, 0)),
            pl.BlockSpec((cm, D), lambda i: (i, 0)),
            pl.BlockSpec((1, cm), lambda i: (0, i)),
            pl.BlockSpec((D, cm), lambda i: (0, i)),
            pl.BlockSpec((B, cm), lambda i: (0, i)),
            pl.BlockSpec((1, D), lambda i: (0, 0)),
        ],
        out_specs=pl.BlockSpec((B, 1, D), lambda i: (0, 0, 0)),
        out_shape=jax.ShapeDtypeStruct((B, 1, D), jnp.float32),
    )(hn, fc1_w, fc1_b.reshape(1, DFF), fc2_w, mask, fc2_b.reshape(1, D))

    return (mlp_out, rout)
